# baseline (speedup 1.0000x reference)
import functools
import math

import jax
import jax.numpy as jnp
from jax import lax
from jax.experimental import pallas as pl
from jax.experimental.pallas import tpu as pltpu
from jax.experimental.pallas import tpu_sc as plsc

_EMBED_DIM = 32
_SEQ_LEN = 200
_BATCH = 4096
_B = _BATCH * _SEQ_LEN
_NW = 32
_SEQ_PER_W = _BATCH // _NW
_CSEQ = 8
_N_CHUNKS = _SEQ_PER_W // _CSEQ
_CHUNK = _CSEQ * _SEQ_LEN
_SCALE = math.sqrt(_EMBED_DIM)
_H = _EMBED_DIM // 2
_L = 16


def _mesh():
    return plsc.VectorSubcoreMesh(core_axis_name="c", subcore_axis_name="s")


@jax.jit
def _tok_embed(x, table, pe):
    @functools.partial(
        pl.kernel,
        mesh=_mesh(),
        compiler_params=pltpu.CompilerParams(use_tc_tiling_on_sc=True),
        out_type=jax.ShapeDtypeStruct((_B,), jnp.int32),
        scratch_types=[
            pltpu.VMEM((_CSEQ, _SEQ_LEN), jnp.int32),
            pltpu.VMEM((_CHUNK,), jnp.int32),
        ],
    )
    def flatten_x(x_hbm, xlin_hbm, xt_v, xf_v):
        wid = lax.axis_index("s") * 2 + lax.axis_index("c")
        row0 = wid * _SEQ_PER_W
        for s in range(_N_CHUNKS):
            pltpu.sync_copy(x_hbm.at[pl.ds(row0 + s * _CSEQ, _CSEQ), :], xt_v)
            for r in range(_CSEQ):
                for c in range(0, _SEQ_LEN - _L + 1, _L):
                    xf_v[pl.ds(r * _SEQ_LEN + c, _L)] = xt_v[r, pl.ds(c, _L)]
                xf_v[pl.ds(r * _SEQ_LEN + _SEQ_LEN - _L, _L)] = xt_v[
                    r, pl.ds(_SEQ_LEN - _L, _L)
                ]
            pltpu.sync_copy(
                xf_v,
                xlin_hbm.at[pl.ds((row0 + s * _CSEQ) * _SEQ_LEN, _CHUNK)],
            )

    xlin = flatten_x(x)

    @functools.partial(
        pl.kernel,
        mesh=_mesh(),
        compiler_params=pltpu.CompilerParams(use_tc_tiling_on_sc=False),
        out_type=jax.ShapeDtypeStruct((_BATCH, _SEQ_LEN + 1, _EMBED_DIM), jnp.float32),
        scratch_types=[
            pltpu.VMEM((_CHUNK,), jnp.int32),
            pltpu.VMEM((_CSEQ, _SEQ_LEN, _EMBED_DIM), jnp.float32),
            pltpu.VMEM((_SEQ_LEN, _EMBED_DIM), jnp.float32),
            pltpu.SemaphoreType.DMA,
        ],
    )
    def k(xlin_hbm, table_hbm, pe_hbm, out_hbm, idx_v, rows_v, pe_v, sem):
        wid = lax.axis_index("s") * 2 + lax.axis_index("c")
        seq_base = wid * _SEQ_PER_W
        pltpu.sync_copy(pe_hbm.at[0, pl.ds(0, _SEQ_LEN), :], pe_v)

        def chunk_body(g, carry):
            s0 = seq_base + g * _CSEQ
            pltpu.sync_copy(xlin_hbm.at[pl.ds(s0 * _SEQ_LEN, _CHUNK)], idx_v)
            descs = [
                pltpu.async_copy(
                    table_hbm.at[idx_v.at[pl.ds(j * _SEQ_LEN, _SEQ_LEN)]],
                    rows_v.at[j],
                    sem,
                )
                for j in range(_CSEQ)
            ]
            for d in descs:
                d.wait()

            def p_body(p, c2):
                pe_lo = pe_v[p, pl.ds(0, _H)]
                pe_hi = pe_v[p, pl.ds(_H, _H)]
                for j in range(_CSEQ):
                    rows_v[j, p, pl.ds(0, _H)] = (
                        rows_v[j, p, pl.ds(0, _H)] * _SCALE + pe_lo
                    )
                    rows_v[j, p, pl.ds(_H, _H)] = (
                        rows_v[j, p, pl.ds(_H, _H)] * _SCALE + pe_hi
                    )
                return c2

            lax.fori_loop(0, _SEQ_LEN, p_body, carry)
            pltpu.sync_copy(
                rows_v, out_hbm.at[pl.ds(s0, _CSEQ), pl.ds(0, _SEQ_LEN), :]
            )
            return carry

        lax.fori_loop(0, _N_CHUNKS, chunk_body, 0)

    out_ext = k(xlin, table, pe)
    return lax.slice_in_dim(out_ext, 0, _SEQ_LEN, axis=1)


def kernel(x, table, pe):
    return _tok_embed(x, table, pe)


# speedup vs baseline: 1.0347x; 1.0347x over previous
import functools
import math

import jax
import jax.numpy as jnp
from jax import lax
from jax.experimental import pallas as pl
from jax.experimental.pallas import tpu as pltpu
from jax.experimental.pallas import tpu_sc as plsc

_EMBED_DIM = 32
_PAD = 128
_SEQ_LEN = 200
_SEQ_PAD = 256
_BATCH = 4096
_B = _BATCH * _SEQ_LEN
_NW = 32
_SEQ_PER_W = _BATCH // _NW
_B_PER_W = _B // _NW
_XROWS = 8
_N_XSLABS = _SEQ_PER_W // _XROWS
_CSEQ = 2
_CHUNK = _CSEQ * _SEQ_LEN
_N_CHUNKS = _SEQ_PER_W // _CSEQ
_SCALE = math.sqrt(_EMBED_DIM)
_H = _EMBED_DIM // 2
_L = 16


def _mesh():
    return plsc.VectorSubcoreMesh(core_axis_name="c", subcore_axis_name="s")


@jax.jit
def _tok_embed(x, table, pe):
    @functools.partial(
        pl.kernel,
        mesh=_mesh(),
        compiler_params=pltpu.CompilerParams(use_tc_tiling_on_sc=True),
        out_type=jax.ShapeDtypeStruct((_BATCH, _SEQ_PAD), jnp.int32),
        scratch_types=[
            pltpu.VMEM((_XROWS, _SEQ_LEN), jnp.int32),
            pltpu.VMEM((_XROWS, _SEQ_PAD), jnp.int32),
        ],
    )
    def flatten_x(x_hbm, xpad_hbm, xt_v, xf_v):
        wid = lax.axis_index("s") * 2 + lax.axis_index("c")
        row0 = wid * _SEQ_PER_W
        for s in range(_N_XSLABS):
            pltpu.sync_copy(x_hbm.at[pl.ds(row0 + s * _XROWS, _XROWS), :], xt_v)
            for r in range(_XROWS):
                for c in range(0, _SEQ_LEN - _L + 1, _L):
                    xf_v[r, pl.ds(c, _L)] = xt_v[r, pl.ds(c, _L)]
                xf_v[r, pl.ds(_SEQ_LEN - _L, _L)] = xt_v[r, pl.ds(_SEQ_LEN - _L, _L)]
            pltpu.sync_copy(xf_v, xpad_hbm.at[pl.ds(row0 + s * _XROWS, _XROWS), :])

    xpad = flatten_x(x)

    @functools.partial(
        pl.kernel,
        mesh=_mesh(),
        compiler_params=pltpu.CompilerParams(use_tc_tiling_on_sc=False),
        out_type=jax.ShapeDtypeStruct((_B, _PAD), jnp.float32),
        scratch_types=[
            pltpu.VMEM((_CHUNK,), jnp.int32),
            pltpu.VMEM((_CHUNK, _EMBED_DIM), jnp.float32),
            pltpu.VMEM((_CHUNK, _PAD), jnp.float32),
            pltpu.VMEM((_SEQ_LEN, _EMBED_DIM), jnp.float32),
            pltpu.SemaphoreType.DMA,
        ],
    )
    def k(xpad_hbm, table_hbm, pe_hbm, out_hbm, idx_v, rows_v, rpad_v, pe_v, sem):
        wid = lax.axis_index("s") * 2 + lax.axis_index("c")
        seq_base = wid * _SEQ_PER_W
        pltpu.sync_copy(pe_hbm.at[0, pl.ds(0, _SEQ_LEN), :], pe_v)

        def chunk_body(g, carry):
            s0 = seq_base + g * _CSEQ
            for j in range(_CSEQ):
                pltpu.sync_copy(
                    xpad_hbm.at[s0 + j, pl.ds(0, _SEQ_LEN)],
                    idx_v.at[pl.ds(j * _SEQ_LEN, _SEQ_LEN)],
                )
            descs = [
                pltpu.async_copy(
                    table_hbm.at[idx_v.at[pl.ds(j * _SEQ_LEN, _SEQ_LEN)]],
                    rows_v.at[pl.ds(j * _SEQ_LEN, _SEQ_LEN), :],
                    sem,
                )
                for j in range(_CSEQ)
            ]
            for d in descs:
                d.wait()

            def p_body(p, c2):
                pe_lo = pe_v[p, pl.ds(0, _H)]
                pe_hi = pe_v[p, pl.ds(_H, _H)]
                for j in range(_CSEQ):
                    r = j * _SEQ_LEN + p
                    rpad_v[r, pl.ds(0, _H)] = (
                        rows_v[r, pl.ds(0, _H)] * _SCALE + pe_lo
                    )
                    rpad_v[r, pl.ds(_H, _H)] = (
                        rows_v[r, pl.ds(_H, _H)] * _SCALE + pe_hi
                    )
                return c2

            lax.fori_loop(0, _SEQ_LEN, p_body, carry)
            pltpu.sync_copy(
                rpad_v,
                out_hbm.at[pl.ds((seq_base + g * _CSEQ) * _SEQ_LEN, _CHUNK), :],
            )
            return carry

        lax.fori_loop(0, _N_CHUNKS, chunk_body, 0)

    out_pad = k(xpad, table, pe)
    out = out_pad.reshape(_BATCH, _SEQ_LEN, _PAD)
    return lax.slice_in_dim(out, 0, _EMBED_DIM, axis=2)


def kernel(x, table, pe):
    return _tok_embed(x, table, pe)


# speedup vs baseline: 1.2207x; 1.1797x over previous
import functools
import math

import jax
import jax.numpy as jnp
from jax import lax
from jax.experimental import pallas as pl
from jax.experimental.pallas import tpu as pltpu
from jax.experimental.pallas import tpu_sc as plsc

_EMBED_DIM = 32
_PAD = 128
_SEQ_LEN = 200
_SEQ_SLOTS = 256
_BATCH = 4096
_B = _BATCH * _SEQ_LEN
_NW = 32
_SEQ_PER_W = _BATCH // _NW
_XROWS = 8
_N_XSLABS = _SEQ_PER_W // _XROWS
_CSEQ = 8
_CHUNK = _CSEQ * _SEQ_LEN
_N_CHUNKS = _SEQ_PER_W // _CSEQ
_SCALE = math.sqrt(_EMBED_DIM)
_H = _EMBED_DIM // 2
_L = 16
_TAIL = _SEQ_LEN - _PAD


def _mesh():
    return plsc.VectorSubcoreMesh(core_axis_name="c", subcore_axis_name="s")


@jax.jit
def _tok_embed(x, table, pe):
    @functools.partial(
        pl.kernel,
        mesh=_mesh(),
        compiler_params=pltpu.CompilerParams(use_tc_tiling_on_sc=True),
        out_type=jax.ShapeDtypeStruct((_BATCH * 2, _PAD), jnp.int32),
        scratch_types=[
            pltpu.VMEM((_XROWS, _SEQ_LEN), jnp.int32),
            pltpu.VMEM((_XROWS * 2, _PAD), jnp.int32),
        ],
    )
    def flatten_x(x_hbm, xpad_hbm, xt_v, xf_v):
        wid = lax.axis_index("s") * 2 + lax.axis_index("c")
        row0 = wid * _SEQ_PER_W
        for s in range(_N_XSLABS):
            pltpu.sync_copy(x_hbm.at[pl.ds(row0 + s * _XROWS, _XROWS), :], xt_v)
            for r in range(_XROWS):
                base = r * _SEQ_SLOTS
                for c in range(0, _SEQ_LEN - _L + 1, _L):
                    o = base + c
                    xf_v[o // _PAD, pl.ds(o % _PAD, _L)] = xt_v[r, pl.ds(c, _L)]
                o = base + _SEQ_LEN - _L
                xf_v[o // _PAD, pl.ds(o % _PAD, _L)] = xt_v[
                    r, pl.ds(_SEQ_LEN - _L, _L)
                ]
            pltpu.sync_copy(
                xf_v, xpad_hbm.at[pl.ds((row0 + s * _XROWS) * 2, _XROWS * 2), :]
            )

    xpad = flatten_x(x)

    @functools.partial(
        pl.kernel,
        mesh=_mesh(),
        compiler_params=pltpu.CompilerParams(use_tc_tiling_on_sc=False),
        out_type=jax.ShapeDtypeStruct((_B, _EMBED_DIM), jnp.float32),
        scratch_types=[
            pltpu.VMEM((_CSEQ * 2, _PAD), jnp.int32),
            pltpu.VMEM((_CHUNK, _EMBED_DIM), jnp.float32),
            pltpu.VMEM((_SEQ_LEN, _EMBED_DIM), jnp.float32),
            pltpu.SemaphoreType.DMA,
        ],
    )
    def k(xpad_hbm, table_hbm, pe_hbm, out_hbm, idx_v, rows_v, pe_v, sem):
        wid = lax.axis_index("s") * 2 + lax.axis_index("c")
        seq_base = wid * _SEQ_PER_W
        pltpu.sync_copy(pe_hbm.at[0, pl.ds(0, _SEQ_LEN), :], pe_v)

        def chunk_body(g, carry):
            s0 = seq_base + g * _CSEQ
            pltpu.sync_copy(xpad_hbm.at[pl.ds(s0 * 2, _CSEQ * 2), :], idx_v)
            descs = []
            for j in range(_CSEQ):
                descs.append(
                    pltpu.async_copy(
                        table_hbm.at[idx_v.at[2 * j, pl.ds(0, _PAD)]],
                        rows_v.at[pl.ds(j * _SEQ_LEN, _PAD), :],
                        sem,
                    )
                )
                descs.append(
                    pltpu.async_copy(
                        table_hbm.at[idx_v.at[2 * j + 1, pl.ds(0, _TAIL)]],
                        rows_v.at[pl.ds(j * _SEQ_LEN + _PAD, _TAIL), :],
                        sem,
                    )
                )
            for d in descs:
                d.wait()

            def p_body(p, c2):
                pe_lo = pe_v[p, pl.ds(0, _H)]
                pe_hi = pe_v[p, pl.ds(_H, _H)]
                for j in range(_CSEQ):
                    r = j * _SEQ_LEN + p
                    rows_v[r, pl.ds(0, _H)] = (
                        rows_v[r, pl.ds(0, _H)] * _SCALE + pe_lo
                    )
                    rows_v[r, pl.ds(_H, _H)] = (
                        rows_v[r, pl.ds(_H, _H)] * _SCALE + pe_hi
                    )
                return c2

            lax.fori_loop(0, _SEQ_LEN, p_body, carry)
            pltpu.sync_copy(
                rows_v, out_hbm.at[pl.ds((seq_base + g * _CSEQ) * _SEQ_LEN, _CHUNK), :]
            )
            return carry

        lax.fori_loop(0, _N_CHUNKS, chunk_body, 0)

    out2d = k(xpad, table, pe)
    return out2d.reshape(_BATCH, _SEQ_LEN, _EMBED_DIM)


def kernel(x, table, pe):
    return _tok_embed(x, table, pe)


# speedup vs baseline: 1.2221x; 1.0012x over previous
import functools
import math

import jax
import jax.numpy as jnp
from jax import lax
from jax.experimental import pallas as pl
from jax.experimental.pallas import tpu as pltpu
from jax.experimental.pallas import tpu_sc as plsc

_EMBED_DIM = 32
_SEQ_LEN = 200
_BATCH = 4096
_NW = 32
_SEQ_PER_W = _BATCH // _NW
_CSEQ = 8
_N_CHUNKS = _SEQ_PER_W // _CSEQ
_SCALE = math.sqrt(_EMBED_DIM)
_H = _EMBED_DIM // 2


@jax.jit
def _tok_embed(x, table, pe):
    mesh = plsc.VectorSubcoreMesh(core_axis_name="c", subcore_axis_name="s")

    @functools.partial(
        pl.kernel,
        mesh=mesh,
        compiler_params=pltpu.CompilerParams(use_tc_tiling_on_sc=False),
        out_type=jax.ShapeDtypeStruct((_BATCH, _SEQ_LEN, _EMBED_DIM), jnp.float32),
        scratch_types=[
            pltpu.VMEM((_CSEQ, _SEQ_LEN), jnp.int32),
            pltpu.VMEM((_CSEQ, _SEQ_LEN, _EMBED_DIM), jnp.float32),
            pltpu.VMEM((_SEQ_LEN, _EMBED_DIM), jnp.float32),
            pltpu.SemaphoreType.DMA,
        ],
    )
    def k(x_hbm, table_hbm, pe_hbm, out_hbm, idx_v, rows_v, pe_v, sem):
        wid = lax.axis_index("s") * 2 + lax.axis_index("c")
        seq_base = wid * _SEQ_PER_W
        pltpu.sync_copy(pe_hbm.at[0, pl.ds(0, _SEQ_LEN), :], pe_v)

        def chunk_body(g, carry):
            s0 = seq_base + g * _CSEQ
            pltpu.sync_copy(x_hbm.at[pl.ds(s0, _CSEQ), :], idx_v)
            descs = [
                pltpu.async_copy(table_hbm.at[idx_v.at[j]], rows_v.at[j], sem)
                for j in range(_CSEQ)
            ]
            for d in descs:
                d.wait()

            def p_body(p, c2):
                pe_lo = pe_v[p, pl.ds(0, _H)]
                pe_hi = pe_v[p, pl.ds(_H, _H)]
                for j in range(_CSEQ):
                    rows_v[j, p, pl.ds(0, _H)] = (
                        rows_v[j, p, pl.ds(0, _H)] * _SCALE + pe_lo
                    )
                    rows_v[j, p, pl.ds(_H, _H)] = (
                        rows_v[j, p, pl.ds(_H, _H)] * _SCALE + pe_hi
                    )
                return c2

            lax.fori_loop(0, _SEQ_LEN, p_body, carry)
            pltpu.sync_copy(rows_v, out_hbm.at[pl.ds(s0, _CSEQ), :, :])
            return carry

        lax.fori_loop(0, _N_CHUNKS, chunk_body, 0)

    return k(x, table, pe)


def kernel(x, table, pe):
    return _tok_embed(x, table, pe)


# speedup vs baseline: 1.2403x; 1.0148x over previous
import functools
import math

import jax
import jax.numpy as jnp
from jax import lax
from jax.experimental import pallas as pl
from jax.experimental.pallas import tpu as pltpu
from jax.experimental.pallas import tpu_sc as plsc

_EMBED_DIM = 32
_SEQ_LEN = 200
_BATCH = 4096
_NW = 32
_SEQ_PER_W = _BATCH // _NW
_CSEQ = 16
_N_CHUNKS = _SEQ_PER_W // _CSEQ
_SCALE = math.sqrt(_EMBED_DIM)
_H = _EMBED_DIM // 2


@jax.jit
def _tok_embed(x, table, pe):
    mesh = plsc.VectorSubcoreMesh(core_axis_name="c", subcore_axis_name="s")

    @functools.partial(
        pl.kernel,
        mesh=mesh,
        compiler_params=pltpu.CompilerParams(use_tc_tiling_on_sc=False),
        out_type=jax.ShapeDtypeStruct((_BATCH, _SEQ_LEN, _EMBED_DIM), jnp.float32),
        scratch_types=[
            pltpu.VMEM((_CSEQ, _SEQ_LEN), jnp.int32),
            pltpu.VMEM((_CSEQ, _SEQ_LEN, _EMBED_DIM), jnp.float32),
            pltpu.VMEM((_SEQ_LEN, _EMBED_DIM), jnp.float32),
            pltpu.SemaphoreType.DMA,
        ],
    )
    def k(x_hbm, table_hbm, pe_hbm, out_hbm, idx_v, rows_v, pe_v, sem):
        wid = lax.axis_index("s") * 2 + lax.axis_index("c")
        seq_base = wid * _SEQ_PER_W
        pltpu.sync_copy(pe_hbm.at[0, pl.ds(0, _SEQ_LEN), :], pe_v)

        def chunk_body(g, carry):
            s0 = seq_base + g * _CSEQ
            pltpu.sync_copy(x_hbm.at[pl.ds(s0, _CSEQ), :], idx_v)
            descs = [
                pltpu.async_copy(table_hbm.at[idx_v.at[j]], rows_v.at[j], sem)
                for j in range(_CSEQ)
            ]
            for d in descs:
                d.wait()

            def p_body(p, c2):
                pe_lo = pe_v[p, pl.ds(0, _H)]
                pe_hi = pe_v[p, pl.ds(_H, _H)]
                for j in range(_CSEQ):
                    rows_v[j, p, pl.ds(0, _H)] = (
                        rows_v[j, p, pl.ds(0, _H)] * _SCALE + pe_lo
                    )
                    rows_v[j, p, pl.ds(_H, _H)] = (
                        rows_v[j, p, pl.ds(_H, _H)] * _SCALE + pe_hi
                    )
                return c2

            lax.fori_loop(0, _SEQ_LEN, p_body, carry)
            pltpu.sync_copy(rows_v, out_hbm.at[pl.ds(s0, _CSEQ), :, :])
            return carry

        lax.fori_loop(0, _N_CHUNKS, chunk_body, 0)

    return k(x, table, pe)


def kernel(x, table, pe):
    return _tok_embed(x, table, pe)
